# baseline (speedup 1.0000x reference)
import functools

import jax
import jax.numpy as jnp
from jax import lax
from jax.experimental import pallas as pl
from jax.experimental.pallas import tpu as pltpu
from jax.experimental.pallas import tpu_sc as plsc

NC = 2
NS = 16
NW = NC * NS

BATCH = 4096
HIST = 200
D = 32
BPW = BATCH // NW
GB = 4
NBUF = 4
NGRP = BPW // GB
NRING = NGRP // NBUF


def _build():
  mesh = plsc.VectorSubcoreMesh(core_axis_name="c", subcore_axis_name="s")

  scratch = (
      [pltpu.VMEM((GB, HIST), jnp.int32) for _ in range(NBUF)]
      + [pltpu.VMEM((GB * HIST, D), jnp.float32) for _ in range(NBUF)]
      + [pltpu.SemaphoreType.DMA for _ in range(3 * NBUF)]
  )

  @functools.partial(
      pl.kernel,
      mesh=mesh,
      out_type=jax.ShapeDtypeStruct((BATCH, HIST, D), jnp.float32),
      scratch_types=scratch,
      compiler_params=pltpu.CompilerParams(use_tc_tiling_on_sc=False),
  )
  def lookup(table_hbm, x_hbm, out_hbm, *refs):
    idx_v = refs[0:NBUF]
    rows_v = refs[NBUF:2 * NBUF]
    idx_sem = refs[2 * NBUF:3 * NBUF]
    g_sem = refs[3 * NBUF:4 * NBUF]
    st_sem = refs[4 * NBUF:5 * NBUF]

    wid = lax.axis_index("s") * NC + lax.axis_index("c")
    base = wid * BPW

    def idx_start(g, s):
      pltpu.async_copy(x_hbm.at[pl.ds(base + g * GB, GB), :],
                       idx_v[s], idx_sem[s])

    def idx_wait(s):
      pltpu.make_async_copy(x_hbm.at[pl.ds(0, GB), :],
                            idx_v[s], idx_sem[s]).wait()

    def gather_start(s):
      for j in range(GB):
        pltpu.async_copy(table_hbm.at[idx_v[s].at[j]],
                         rows_v[s].at[pl.ds(j * HIST, HIST), :], g_sem[s])

    def gather_wait(s):
      for j in range(GB):
        pltpu.make_async_copy(table_hbm.at[idx_v[s].at[j]],
                              rows_v[s].at[pl.ds(j * HIST, HIST), :],
                              g_sem[s]).wait()

    def store_start(g, s):
      for j in range(GB):
        pltpu.async_copy(rows_v[s].at[pl.ds(j * HIST, HIST), :],
                         out_hbm.at[base + g * GB + j], st_sem[s])

    def store_wait(s):
      for j in range(GB):
        pltpu.make_async_copy(rows_v[s].at[pl.ds(j * HIST, HIST), :],
                              out_hbm.at[0], st_sem[s]).wait()

    for s in range(NBUF):
      idx_start(s, s)

    def ring(r, carry):
      for b in range(NBUF):
        g = r * NBUF + b
        idx_wait(b)
        pl.when(r >= 1)(lambda: store_wait(b))
        gather_start(b)
        bp = (b - 1) % NBUF
        if b == 0:
          def retire_prev_ring():
            gather_wait(bp)
            store_start(g - 1, bp)
            idx_start(g + NBUF - 1, bp)
          pl.when(r >= 1)(retire_prev_ring)
        else:
          gather_wait(bp)
          store_start(g - 1, bp)
          pl.when(r < NRING - 1)(
              functools.partial(idx_start, g + NBUF - 1, bp))
      return carry

    lax.fori_loop(0, NRING, ring, 0)

    s_last = (NGRP - 1) % NBUF
    gather_wait(s_last)
    store_start(NGRP - 1, s_last)
    for s in range(NBUF):
      store_wait(s)

  return lookup


_lookup = _build()


@jax.jit
def kernel(X, W):
  return _lookup(W, X)
